# Initial kernel scaffold; baseline (speedup 1.0000x reference)
import jax
import jax.numpy as jnp
from jax.experimental import pallas as pl


def kernel(roi_features, union_features, rel_pair_idxs, obj_labels, W_obj_in, W_rel_in, W_s2r, W_o2r, W_r2s, W_r2o, W_obj_cls, b_obj, W_rel_cls, b_rel, freq_table):
    raise NotImplementedError("write your pallas kernel here")


# speedup vs baseline: 1.4304x; 1.4304x over previous
import functools

import jax
import jax.numpy as jnp
from jax import lax
from jax.experimental import pallas as pl
from jax.experimental.pallas import tpu as pltpu
from jax.experimental.pallas import tpu_sc as plsc

_L = 16
_NC = 2
_NS = 16
_NW = _NC * _NS
_EB = 128


def _sc_mesh():
    return plsc.VectorSubcoreMesh(
        core_axis_name="c", subcore_axis_name="s",
        num_cores=_NC, num_subcores=_NS)


_SC_PARAMS = pltpu.CompilerParams(needs_layout_passes=False)


def _cdiv(a, b):
    return (a + b - 1) // b


def _mm_body(x_ref, w_ref, o_ref, *, relu, b_ref=None):
    acc = jnp.dot(x_ref[...], w_ref[...], preferred_element_type=jnp.float32)
    if b_ref is not None:
        acc = acc + b_ref[...]
    o_ref[...] = jnp.maximum(acc, 0.0) if relu else acc


def _mm(x, w, *, out_rows=None, relu=False, bias=None, bm=512):
    m, k = x.shape
    n = w.shape[1]
    mo = m if out_rows is None else out_rows
    nxb = _cdiv(m, bm)
    in_specs = [
        pl.BlockSpec((bm, k), lambda i: (jnp.minimum(i, nxb - 1), 0)),
        pl.BlockSpec((k, n), lambda i: (0, 0)),
    ]
    args = [x, w]
    if bias is not None:
        in_specs.append(pl.BlockSpec((1, n), lambda i: (0, 0)))
        args.append(bias)
        body = lambda xr, wr, br, orf: _mm_body(xr, wr, orf, relu=relu, b_ref=br)
    else:
        body = functools.partial(_mm_body, relu=relu)
    return pl.pallas_call(
        body,
        grid=(_cdiv(mo, bm),),
        in_specs=in_specs,
        out_specs=pl.BlockSpec((bm, n), lambda i: (i, 0)),
        out_shape=jax.ShapeDtypeStruct((mo, n), jnp.float32),
    )(*args)


def _proj_body(x_ref, ws_ref, wo_ref, ps_ref, po_ref):
    x = x_ref[...]
    ps_ref[0] = jnp.dot(x, ws_ref[0], preferred_element_type=jnp.float32)
    po_ref[0] = jnp.dot(x, wo_ref[0], preferred_element_type=jnp.float32)


def _proj(obj_h, w_s, w_o, np_, nch, wc, bm=512):
    n, h = obj_h.shape
    nxb = _cdiv(n, bm)
    return pl.pallas_call(
        _proj_body,
        grid=(nch, _cdiv(np_, bm)),
        in_specs=[
            pl.BlockSpec((bm, h), lambda c, i: (jnp.minimum(i, nxb - 1), 0)),
            pl.BlockSpec((1, h, wc), lambda c, i: (c, 0, 0)),
            pl.BlockSpec((1, h, wc), lambda c, i: (c, 0, 0)),
        ],
        out_specs=[
            pl.BlockSpec((1, bm, wc), lambda c, i: (c, i, 0)),
            pl.BlockSpec((1, bm, wc), lambda c, i: (c, i, 0)),
        ],
        out_shape=[
            jax.ShapeDtypeStruct((nch, np_, wc), jnp.float32),
            jax.ShapeDtypeStruct((nch, np_, wc), jnp.float32),
        ],
    )(obj_h, w_s, w_o)


def _objupd_body(oh_ref, ss_ref, so_ref, ids_ref, ido_ref,
                 wrs_ref, wro_ref, o_ref):
    acc = oh_ref[...]
    acc = acc + jnp.dot(ss_ref[...] * ids_ref[...], wrs_ref[...],
                        preferred_element_type=jnp.float32)
    acc = acc + jnp.dot(so_ref[...] * ido_ref[...], wro_ref[...],
                        preferred_element_type=jnp.float32)
    o_ref[...] = jnp.maximum(acc, 0.0)


def _objupd(obj_h, s_s, s_o, ids, ido, w_rs, w_ro, bm=512):
    n, h = obj_h.shape
    return pl.pallas_call(
        _objupd_body,
        grid=(_cdiv(n, bm),),
        in_specs=[
            pl.BlockSpec((bm, h), lambda i: (i, 0)),
            pl.BlockSpec((bm, h), lambda i: (i, 0)),
            pl.BlockSpec((bm, h), lambda i: (i, 0)),
            pl.BlockSpec((bm, 1), lambda i: (i, 0)),
            pl.BlockSpec((bm, 1), lambda i: (i, 0)),
            pl.BlockSpec((h, h), lambda i: (0, 0)),
            pl.BlockSpec((h, h), lambda i: (0, 0)),
        ],
        out_specs=pl.BlockSpec((bm, h), lambda i: (i, 0)),
        out_shape=jax.ShapeDtypeStruct((n, h), jnp.float32),
    )(obj_h, s_s, s_o, ids, ido, w_rs, w_ro)


def _rellog_body(x_ref, w_ref, b_ref, fb_ref, o_ref):
    acc = jnp.dot(x_ref[...], w_ref[...], preferred_element_type=jnp.float32)
    o_ref[...] = acc + b_ref[...] + fb_ref[...]


def _rellog(rel_h, w, b, fbias, e, bm=1024):
    h = rel_h.shape[1]
    n = w.shape[1]
    return pl.pallas_call(
        _rellog_body,
        grid=(_cdiv(e, bm),),
        in_specs=[
            pl.BlockSpec((bm, h), lambda i: (i, 0)),
            pl.BlockSpec((h, n), lambda i: (0, 0)),
            pl.BlockSpec((1, n), lambda i: (0, 0)),
            pl.BlockSpec((bm, n), lambda i: (i, 0)),
        ],
        out_specs=pl.BlockSpec((bm, n), lambda i: (i, 0)),
        out_shape=jax.ShapeDtypeStruct((e, n), jnp.float32),
    )(rel_h, w, b, fbias)


def _dega_build(np_, ep, epw, nblk, nobj, nbias):
    def body(sub_hbm, obj_hbm, labels_hbm, freq_hbm,
             ds_hbm, do_hbm, bias_hbm,
             labels_v, degs_v, dego_v, subv, objv, pairv, biasv, sem):
        c = lax.axis_index("c")
        s = lax.axis_index("s")
        wid = c * _NS + s
        pltpu.sync_copy(labels_hbm, labels_v)
        zeros16 = jnp.zeros((_L,), jnp.float32)
        ones16 = jnp.ones((_L,), jnp.float32)

        def zbody(j, carry):
            degs_v[pl.ds(j * _L, _L)] = zeros16
            dego_v[pl.ds(j * _L, _L)] = zeros16
            return carry
        lax.fori_loop(0, np_ // _L, zbody, 0)

        def blk(b, carry):
            base = wid * epw + b * _EB
            pltpu.sync_copy(sub_hbm.at[pl.ds(base, _EB)], subv)
            pltpu.sync_copy(obj_hbm.at[pl.ds(base, _EB)], objv)

            def lane(k, carry2):
                si = subv[pl.ds(k * _L, _L)]
                oi = objv[pl.ds(k * _L, _L)]
                plsc.addupdate_scatter(degs_v, [si], ones16)
                plsc.addupdate_scatter(dego_v, [oi], ones16)
                ls = plsc.load_gather(labels_v, [si])
                lo = plsc.load_gather(labels_v, [oi])
                pairv[pl.ds(k * _L, _L)] = ls * nobj + lo
                return carry2
            lax.fori_loop(0, _EB // _L, lane, 0)
            pltpu.async_copy(freq_hbm.at[pairv], biasv, sem).wait()
            pltpu.sync_copy(biasv, bias_hbm.at[pl.ds(base, _EB)])
            return carry
        lax.fori_loop(0, nblk, blk, 0)
        pltpu.sync_copy(degs_v, ds_hbm.at[wid])
        pltpu.sync_copy(dego_v, do_hbm.at[wid])

    return pl.kernel(
        body,
        out_type=[
            jax.ShapeDtypeStruct((_NW, np_), jnp.float32),
            jax.ShapeDtypeStruct((_NW, np_), jnp.float32),
            jax.ShapeDtypeStruct((ep, nbias), jnp.float32),
        ],
        mesh=_sc_mesh(),
        compiler_params=_SC_PARAMS,
        scratch_types=[
            pltpu.VMEM((np_,), jnp.int32),
            pltpu.VMEM((np_,), jnp.float32),
            pltpu.VMEM((np_,), jnp.float32),
            pltpu.VMEM((_EB,), jnp.int32),
            pltpu.VMEM((_EB,), jnp.int32),
            pltpu.VMEM((_EB,), jnp.int32),
            pltpu.VMEM((_EB, nbias), jnp.float32),
            pltpu.SemaphoreType.DMA,
        ],
    )


def _inv_build(np_):
    stripe = 512
    nact = np_ // stripe

    def body(ds_hbm, do_hbm, invs_hbm, invo_hbm, d_v, out_v):
        c = lax.axis_index("c")
        s = lax.axis_index("s")
        wid = c * _NS + s
        r0 = wid * stripe

        @pl.when(wid < nact)
        def _():
            def one(d_hbm, inv_hbm):
                pltpu.sync_copy(
                    d_hbm.at[pl.ds(0, _NW), pl.ds(r0, stripe)], d_v)

                def red(j, carry):
                    acc = jnp.zeros((_L,), jnp.float32)
                    for k in range(_NW):
                        acc = acc + d_v[k, pl.ds(j * _L, _L)]
                    out_v[pl.ds(j * _L, _L)] = 1.0 / jnp.maximum(acc, 1.0)
                    return carry
                lax.fori_loop(0, stripe // _L, red, 0)
                pltpu.sync_copy(out_v, inv_hbm.at[pl.ds(r0, stripe)])
            one(ds_hbm, invs_hbm)
            one(do_hbm, invo_hbm)

    return pl.kernel(
        body,
        out_type=[
            jax.ShapeDtypeStruct((np_,), jnp.float32),
            jax.ShapeDtypeStruct((np_,), jnp.float32),
        ],
        mesh=_sc_mesh(),
        compiler_params=_SC_PARAMS,
        scratch_types=[
            pltpu.VMEM((_NW, stripe), jnp.float32),
            pltpu.VMEM((stripe,), jnp.float32),
        ],
    )


def _b1_build(np_, ep, epw, nblk, h, nch, wc):
    kv = wc // _L

    def body(ps_hbm, po_hbm, relh_hbm, sub_hbm, obj_hbm,
             relo_hbm,
             subv, objv, gs_v, go_v, rh_v, sem1, sem2, sem3):
        c = lax.axis_index("c")
        s = lax.axis_index("s")
        wid = c * _NS + s
        e0 = wid * epw

        def blk(b, carry):
            base = e0 + b * _EB
            pltpu.sync_copy(sub_hbm.at[pl.ds(base, _EB)], subv)
            pltpu.sync_copy(obj_hbm.at[pl.ds(base, _EB)], objv)
            for p in range(nch):
                cp1 = pltpu.async_copy(ps_hbm.at[p].at[subv], gs_v, sem1)
                cp2 = pltpu.async_copy(po_hbm.at[p].at[objv], go_v, sem2)
                cp3 = pltpu.async_copy(
                    relh_hbm.at[pl.ds(base, _EB), pl.ds(p * wc, wc)],
                    rh_v, sem3)
                cp1.wait()
                cp2.wait()
                cp3.wait()

                def row(r, carry2):
                    for k in range(kv):
                        sl = pl.ds(k * _L, _L)
                        m = jnp.maximum(gs_v[r, sl] + go_v[r, sl], 0.0)
                        rh_v[r, sl] = jnp.maximum(rh_v[r, sl] + m, 0.0)
                    return carry2
                lax.fori_loop(0, _EB, row, 0)
                pltpu.sync_copy(
                    rh_v, relo_hbm.at[pl.ds(base, _EB), pl.ds(p * wc, wc)])
            return carry
        lax.fori_loop(0, nblk, blk, 0)

    return pl.kernel(
        body,
        out_type=jax.ShapeDtypeStruct((ep, h), jnp.float32),
        mesh=_sc_mesh(),
        compiler_params=_SC_PARAMS,
        scratch_types=[
            pltpu.VMEM((_EB,), jnp.int32),
            pltpu.VMEM((_EB,), jnp.int32),
            pltpu.VMEM((_EB, wc), jnp.float32),
            pltpu.VMEM((_EB, wc), jnp.float32),
            pltpu.VMEM((_EB, wc), jnp.float32),
            pltpu.SemaphoreType.DMA,
            pltpu.SemaphoreType.DMA,
            pltpu.SemaphoreType.DMA,
        ],
    )


def _b2_build(np_, ep, h, nch, wc):
    stripe = np_ // _NS
    piece = stripe // 8
    epc = ep // _NS
    nblk = epc // _EB
    kv = wc // _L

    def body(relh_hbm, sub_hbm, obj_hbm,
             ss_hbm, so_hbm,
             idxv, rh_v, zb_v, a_sh, sem1):
        c = lax.axis_index("c")
        s = lax.axis_index("s")
        e0 = s * epc
        r0 = s * stripe
        zeros16 = jnp.zeros((_L,), jnp.float32)

        def zrow(r, carry):
            for k in range(kv):
                zb_v[r, pl.ds(k * _L, _L)] = zeros16
            return carry
        lax.fori_loop(0, piece, zrow, 0)

        def run(idx_hbm, out_hbm):
            for p in range(nch):
                for q in range(8):
                    pltpu.sync_copy(
                        zb_v, a_sh.at[pl.ds(r0 + q * piece, piece)])
                plsc.subcore_barrier()

                def blk(b, carry):
                    base = e0 + b * _EB
                    pltpu.sync_copy(idx_hbm.at[pl.ds(base, _EB)], idxv)
                    cp = pltpu.async_copy(
                        relh_hbm.at[pl.ds(base, _EB), pl.ds(p * wc, wc)],
                        rh_v, sem1)
                    cp.wait()
                    pltpu.sync_copy(rh_v, a_sh.at[idxv], add=True)
                    return carry
                lax.fori_loop(0, nblk, blk, 0)
                plsc.subcore_barrier()
                for q in range(8):
                    pltpu.sync_copy(
                        a_sh.at[pl.ds(r0 + q * piece, piece)],
                        out_hbm.at[pl.ds(r0 + q * piece, piece),
                                   pl.ds(p * wc, wc)])
                plsc.subcore_barrier()

        @pl.when(c == 0)
        def _():
            run(sub_hbm, ss_hbm)

        @pl.when(c == 1)
        def _():
            run(obj_hbm, so_hbm)

    return pl.kernel(
        body,
        out_type=[
            jax.ShapeDtypeStruct((np_, h), jnp.float32),
            jax.ShapeDtypeStruct((np_, h), jnp.float32),
        ],
        mesh=_sc_mesh(),
        compiler_params=_SC_PARAMS,
        scratch_types=[
            pltpu.VMEM((_EB,), jnp.int32),
            pltpu.VMEM((_EB, wc), jnp.float32),
            pltpu.VMEM((piece, wc), jnp.float32),
            pltpu.VMEM_SHARED((np_, wc), jnp.float32),
            pltpu.SemaphoreType.DMA,
        ],
    )


def kernel(roi_features, union_features, rel_pair_idxs, obj_labels,
           W_obj_in, W_rel_in, W_s2r, W_o2r, W_r2s, W_r2o,
           W_obj_cls, b_obj, W_rel_cls, b_rel, freq_table):
    n, h = roi_features.shape[0], W_obj_in.shape[1]
    e = union_features.shape[0]
    nobj = W_obj_cls.shape[1]
    nrel = W_rel_cls.shape[1]
    iters = 3

    wc = 128
    nch = h // wc
    np_ = ((n + 1 + _NW * _L - 1) // (_NW * _L)) * (_NW * _L)
    epw = _cdiv(e, _NW * _EB) * _EB
    ep = epw * _NW
    nblk = epw // _EB
    nbias = 128

    sub_p = jnp.concatenate(
        [rel_pair_idxs[:, 0], jnp.full((ep - e,), n, jnp.int32)])
    obj_p = jnp.concatenate(
        [rel_pair_idxs[:, 1], jnp.full((ep - e,), n, jnp.int32)])
    labels_p = jnp.concatenate(
        [obj_labels, jnp.zeros((np_ - n,), jnp.int32)])
    freq_p = jnp.pad(freq_table, ((0, 7), (0, nbias - nrel)))
    w_obj_cls_p = jnp.pad(W_obj_cls, ((0, 0), (0, 256 - nobj)))
    b_obj_p = jnp.pad(b_obj, (0, 256 - nobj)).reshape(1, 256)
    w_rel_cls_p = jnp.pad(W_rel_cls, ((0, 0), (0, nbias - nrel)))
    b_rel_p = jnp.pad(b_rel, (0, nbias - nrel)).reshape(1, nbias)

    d_s, d_o, fbias = _dega_build(np_, ep, epw, nblk, nobj, nbias)(
        sub_p, obj_p, labels_p, freq_p)
    inv_ds, inv_do = _inv_build(np_)(d_s, d_o)

    obj_h = _mm(roi_features, W_obj_in, relu=True)
    rel_h = _mm(union_features, W_rel_in, out_rows=ep, relu=True)

    w_s2r_c = W_s2r.reshape(h, nch, wc).transpose(1, 0, 2)
    w_o2r_c = W_o2r.reshape(h, nch, wc).transpose(1, 0, 2)
    b1 = _b1_build(np_, ep, epw, nblk, h, nch, wc)
    b2 = _b2_build(np_, ep, h, nch, wc)
    ids = inv_ds.reshape(np_, 1)
    ido = inv_do.reshape(np_, 1)
    for _ in range(iters):
        p_s, p_o = _proj(obj_h, w_s2r_c, w_o2r_c, np_, nch, wc)
        rel_h = b1(p_s, p_o, rel_h, sub_p, obj_p)
        s_s, s_o = b2(rel_h, sub_p, obj_p)
        obj_h = _objupd(obj_h, s_s, s_o, ids, ido, W_r2s, W_r2o)

    obj_logits = _mm(obj_h, w_obj_cls_p, bias=b_obj_p)[:, :nobj]
    rel_logits = _rellog(rel_h, w_rel_cls_p, b_rel_p, fbias, e)[:, :nrel]
    return obj_logits, rel_logits


# speedup vs baseline: 1.5992x; 1.1180x over previous
import functools

import jax
import jax.numpy as jnp
from jax import lax
from jax.experimental import pallas as pl
from jax.experimental.pallas import tpu as pltpu
from jax.experimental.pallas import tpu_sc as plsc

_L = 16
_NC = 2
_NS = 16
_NW = _NC * _NS
_EB = 128


def _sc_mesh():
    return plsc.VectorSubcoreMesh(
        core_axis_name="c", subcore_axis_name="s",
        num_cores=_NC, num_subcores=_NS)


_SC_PARAMS = pltpu.CompilerParams(needs_layout_passes=False)


def _cdiv(a, b):
    return (a + b - 1) // b


def _mm_body(x_ref, w_ref, o_ref, *, relu, b_ref=None):
    acc = jnp.dot(x_ref[...], w_ref[...], preferred_element_type=jnp.float32)
    if b_ref is not None:
        acc = acc + b_ref[...]
    o_ref[...] = jnp.maximum(acc, 0.0) if relu else acc


def _mm(x, w, *, out_rows=None, relu=False, bias=None, bm=512):
    m, k = x.shape
    n = w.shape[1]
    mo = m if out_rows is None else out_rows
    nxb = _cdiv(m, bm)
    in_specs = [
        pl.BlockSpec((bm, k), lambda i: (jnp.minimum(i, nxb - 1), 0)),
        pl.BlockSpec((k, n), lambda i: (0, 0)),
    ]
    args = [x, w]
    if bias is not None:
        in_specs.append(pl.BlockSpec((1, n), lambda i: (0, 0)))
        args.append(bias)
        body = lambda xr, wr, br, orf: _mm_body(xr, wr, orf, relu=relu, b_ref=br)
    else:
        body = functools.partial(_mm_body, relu=relu)
    return pl.pallas_call(
        body,
        grid=(_cdiv(mo, bm),),
        in_specs=in_specs,
        out_specs=pl.BlockSpec((bm, n), lambda i: (i, 0)),
        out_shape=jax.ShapeDtypeStruct((mo, n), jnp.float32),
    )(*args)


def _proj_body(x_ref, ws_ref, wo_ref, ps_ref, po_ref):
    x = x_ref[...]
    ps_ref[0] = jnp.dot(x, ws_ref[0], preferred_element_type=jnp.float32)
    po_ref[0] = jnp.dot(x, wo_ref[0], preferred_element_type=jnp.float32)


def _proj(obj_h, w_s, w_o, np_, nch, wc, bm=512):
    n, h = obj_h.shape
    nxb = _cdiv(n, bm)
    return pl.pallas_call(
        _proj_body,
        grid=(nch, _cdiv(np_, bm)),
        in_specs=[
            pl.BlockSpec((bm, h), lambda c, i: (jnp.minimum(i, nxb - 1), 0)),
            pl.BlockSpec((1, h, wc), lambda c, i: (c, 0, 0)),
            pl.BlockSpec((1, h, wc), lambda c, i: (c, 0, 0)),
        ],
        out_specs=[
            pl.BlockSpec((1, bm, wc), lambda c, i: (c, i, 0)),
            pl.BlockSpec((1, bm, wc), lambda c, i: (c, i, 0)),
        ],
        out_shape=[
            jax.ShapeDtypeStruct((nch, np_, wc), jnp.float32),
            jax.ShapeDtypeStruct((nch, np_, wc), jnp.float32),
        ],
    )(obj_h, w_s, w_o)


def _objupd_body(oh_ref, ss_ref, so_ref, ids_ref, ido_ref,
                 wrs_ref, wro_ref, o_ref):
    acc = oh_ref[...]
    acc = acc + jnp.dot(ss_ref[...] * ids_ref[...], wrs_ref[...],
                        preferred_element_type=jnp.float32)
    acc = acc + jnp.dot(so_ref[...] * ido_ref[...], wro_ref[...],
                        preferred_element_type=jnp.float32)
    o_ref[...] = jnp.maximum(acc, 0.0)


def _objupd(obj_h, s_s, s_o, ids, ido, w_rs, w_ro, bm=512):
    n, h = obj_h.shape
    return pl.pallas_call(
        _objupd_body,
        grid=(_cdiv(n, bm),),
        in_specs=[
            pl.BlockSpec((bm, h), lambda i: (i, 0)),
            pl.BlockSpec((bm, h), lambda i: (i, 0)),
            pl.BlockSpec((bm, h), lambda i: (i, 0)),
            pl.BlockSpec((bm, 1), lambda i: (i, 0)),
            pl.BlockSpec((bm, 1), lambda i: (i, 0)),
            pl.BlockSpec((h, h), lambda i: (0, 0)),
            pl.BlockSpec((h, h), lambda i: (0, 0)),
        ],
        out_specs=pl.BlockSpec((bm, h), lambda i: (i, 0)),
        out_shape=jax.ShapeDtypeStruct((n, h), jnp.float32),
    )(obj_h, s_s, s_o, ids, ido, w_rs, w_ro)


def _rellog_body(x_ref, w_ref, b_ref, fb_ref, o_ref):
    acc = jnp.dot(x_ref[...], w_ref[...], preferred_element_type=jnp.float32)
    o_ref[...] = acc + b_ref[...] + fb_ref[...]


def _rellog(rel_h, w, b, fbias, e, bm=1024):
    h = rel_h.shape[1]
    n = w.shape[1]
    return pl.pallas_call(
        _rellog_body,
        grid=(_cdiv(e, bm),),
        in_specs=[
            pl.BlockSpec((bm, h), lambda i: (i, 0)),
            pl.BlockSpec((h, n), lambda i: (0, 0)),
            pl.BlockSpec((1, n), lambda i: (0, 0)),
            pl.BlockSpec((bm, n), lambda i: (i, 0)),
        ],
        out_specs=pl.BlockSpec((bm, n), lambda i: (i, 0)),
        out_shape=jax.ShapeDtypeStruct((e, n), jnp.float32),
    )(rel_h, w, b, fbias)


def _dega_build(np_, ep, epw, nblk, nobj, nbias):
    def body(sub_hbm, obj_hbm, labels_hbm, freq_hbm,
             ds_hbm, do_hbm, bias_hbm,
             labels_v, degs_v, dego_v, subv, objv, pairv, biasv, sem):
        c = lax.axis_index("c")
        s = lax.axis_index("s")
        wid = c * _NS + s
        pltpu.sync_copy(labels_hbm, labels_v)
        zeros16 = jnp.zeros((_L,), jnp.float32)
        ones16 = jnp.ones((_L,), jnp.float32)

        def zbody(j, carry):
            degs_v[pl.ds(j * _L, _L)] = zeros16
            dego_v[pl.ds(j * _L, _L)] = zeros16
            return carry
        lax.fori_loop(0, np_ // _L, zbody, 0)

        def blk(b, carry):
            base = wid * epw + b * _EB
            pltpu.sync_copy(sub_hbm.at[pl.ds(base, _EB)], subv)
            pltpu.sync_copy(obj_hbm.at[pl.ds(base, _EB)], objv)

            def lane(k, carry2):
                si = subv[pl.ds(k * _L, _L)]
                oi = objv[pl.ds(k * _L, _L)]
                plsc.addupdate_scatter(degs_v, [si], ones16)
                plsc.addupdate_scatter(dego_v, [oi], ones16)
                ls = plsc.load_gather(labels_v, [si])
                lo = plsc.load_gather(labels_v, [oi])
                pairv[pl.ds(k * _L, _L)] = ls * nobj + lo
                return carry2
            lax.fori_loop(0, _EB // _L, lane, 0)
            pltpu.async_copy(freq_hbm.at[pairv], biasv, sem).wait()
            pltpu.sync_copy(biasv, bias_hbm.at[pl.ds(base, _EB)])
            return carry
        lax.fori_loop(0, nblk, blk, 0)
        pltpu.sync_copy(degs_v, ds_hbm.at[wid])
        pltpu.sync_copy(dego_v, do_hbm.at[wid])

    return pl.kernel(
        body,
        out_type=[
            jax.ShapeDtypeStruct((_NW, np_), jnp.float32),
            jax.ShapeDtypeStruct((_NW, np_), jnp.float32),
            jax.ShapeDtypeStruct((ep, nbias), jnp.float32),
        ],
        mesh=_sc_mesh(),
        compiler_params=_SC_PARAMS,
        scratch_types=[
            pltpu.VMEM((np_,), jnp.int32),
            pltpu.VMEM((np_,), jnp.float32),
            pltpu.VMEM((np_,), jnp.float32),
            pltpu.VMEM((_EB,), jnp.int32),
            pltpu.VMEM((_EB,), jnp.int32),
            pltpu.VMEM((_EB,), jnp.int32),
            pltpu.VMEM((_EB, nbias), jnp.float32),
            pltpu.SemaphoreType.DMA,
        ],
    )


def _inv_build(np_):
    stripe = 512
    nact = np_ // stripe

    def body(ds_hbm, do_hbm, invs_hbm, invo_hbm, d_v, out_v):
        c = lax.axis_index("c")
        s = lax.axis_index("s")
        wid = c * _NS + s
        r0 = wid * stripe

        @pl.when(wid < nact)
        def _():
            def one(d_hbm, inv_hbm):
                pltpu.sync_copy(
                    d_hbm.at[pl.ds(0, _NW), pl.ds(r0, stripe)], d_v)

                def red(j, carry):
                    acc = jnp.zeros((_L,), jnp.float32)
                    for k in range(_NW):
                        acc = acc + d_v[k, pl.ds(j * _L, _L)]
                    out_v[pl.ds(j * _L, _L)] = 1.0 / jnp.maximum(acc, 1.0)
                    return carry
                lax.fori_loop(0, stripe // _L, red, 0)
                pltpu.sync_copy(out_v, inv_hbm.at[pl.ds(r0, stripe)])
            one(ds_hbm, invs_hbm)
            one(do_hbm, invo_hbm)

    return pl.kernel(
        body,
        out_type=[
            jax.ShapeDtypeStruct((np_,), jnp.float32),
            jax.ShapeDtypeStruct((np_,), jnp.float32),
        ],
        mesh=_sc_mesh(),
        compiler_params=_SC_PARAMS,
        scratch_types=[
            pltpu.VMEM((_NW, stripe), jnp.float32),
            pltpu.VMEM((stripe,), jnp.float32),
        ],
    )


def _b1_build(np_, ep, epw, nblk, h, nch, wc):
    kv = wc // _L

    def body(ps_hbm, po_hbm, relh_hbm, sub3_hbm, obj3_hbm,
             relo_hbm,
             subs_v, objs_v, gs0, go0, rh0, gs1, go1, rh1,
             semr0, semr1, semw0, semw1):
        c = lax.axis_index("c")
        s = lax.axis_index("s")
        wid = c * _NS + s
        e0 = wid * epw
        brow = wid * nblk
        pltpu.sync_copy(sub3_hbm.at[pl.ds(brow, nblk)], subs_v)
        pltpu.sync_copy(obj3_hbm.at[pl.ds(brow, nblk)], objs_v)
        bufs = ((gs0, go0, rh0, semr0, semw0),
                (gs1, go1, rh1, semr1, semw1))

        for p in range(nch):
            col = p * wc

            def reads(b_, sl):
                gs, go, rh, semr, _ = bufs[sl]
                base = e0 + b_ * _EB
                c1 = pltpu.async_copy(ps_hbm.at[p].at[subs_v.at[b_]],
                                      gs, semr)
                c2 = pltpu.async_copy(po_hbm.at[p].at[objs_v.at[b_]],
                                      go, semr)
                c3 = pltpu.async_copy(
                    relh_hbm.at[pl.ds(base, _EB), pl.ds(col, wc)], rh, semr)
                return c1, c2, c3

            def wait_reads(sl):
                gs, go, rh, semr, _ = bufs[sl]
                pltpu.make_async_copy(
                    ps_hbm.at[p].at[subs_v.at[0]], gs, semr).wait()
                pltpu.make_async_copy(
                    po_hbm.at[p].at[objs_v.at[0]], go, semr).wait()
                pltpu.make_async_copy(
                    relh_hbm.at[pl.ds(e0, _EB), pl.ds(col, wc)],
                    rh, semr).wait()

            def wait_write(sl):
                gs, go, rh, _, semw = bufs[sl]
                pltpu.make_async_copy(
                    rh, relo_hbm.at[pl.ds(e0, _EB), pl.ds(col, wc)],
                    semw).wait()

            reads(0, 0)

            def pair(g, carry):
                for sl in range(2):
                    b_ = g * 2 + sl
                    wait_reads(sl)

                    @pl.when(b_ + 1 < nblk)
                    def _():
                        @pl.when(b_ >= 1)
                        def _():
                            wait_write(1 - sl)
                        reads(b_ + 1, 1 - sl)

                    gs, go, rh, _, semw = bufs[sl]

                    def row(r, carry2):
                        for k in range(kv):
                            slc = pl.ds(k * _L, _L)
                            m = jnp.maximum(gs[r, slc] + go[r, slc], 0.0)
                            rh[r, slc] = jnp.maximum(rh[r, slc] + m, 0.0)
                        return carry2
                    lax.fori_loop(0, _EB, row, 0)
                    base = e0 + b_ * _EB
                    pltpu.async_copy(
                        rh, relo_hbm.at[pl.ds(base, _EB), pl.ds(col, wc)],
                        semw)
                return carry
            lax.fori_loop(0, nblk // 2, pair, 0)
            wait_write(0)
            wait_write(1)

    return pl.kernel(
        body,
        out_type=jax.ShapeDtypeStruct((ep, h), jnp.float32),
        mesh=_sc_mesh(),
        compiler_params=_SC_PARAMS,
        scratch_types=[
            pltpu.VMEM((nblk, _EB), jnp.int32),
            pltpu.VMEM((nblk, _EB), jnp.int32),
            pltpu.VMEM((_EB, wc), jnp.float32),
            pltpu.VMEM((_EB, wc), jnp.float32),
            pltpu.VMEM((_EB, wc), jnp.float32),
            pltpu.VMEM((_EB, wc), jnp.float32),
            pltpu.VMEM((_EB, wc), jnp.float32),
            pltpu.VMEM((_EB, wc), jnp.float32),
            pltpu.SemaphoreType.DMA,
            pltpu.SemaphoreType.DMA,
            pltpu.SemaphoreType.DMA,
            pltpu.SemaphoreType.DMA,
        ],
    )


def _b2_build(np_, ep, h, nch, wc):
    stripe = np_ // _NS
    piece = stripe // 8
    epc = ep // _NS
    nblk = epc // _EB
    kv = wc // _L

    def body(relh_hbm, sub3_hbm, obj3_hbm,
             ss_hbm, so_hbm,
             idxs_v, rh_v, zb_v, a_sh, sem1):
        c = lax.axis_index("c")
        s = lax.axis_index("s")
        e0 = s * epc
        r0 = s * stripe
        brow = s * nblk
        zeros16 = jnp.zeros((_L,), jnp.float32)

        def zrow(r, carry):
            for k in range(kv):
                zb_v[r, pl.ds(k * _L, _L)] = zeros16
            return carry
        lax.fori_loop(0, piece, zrow, 0)

        def run(idx3_hbm, out_hbm):
            pltpu.sync_copy(idx3_hbm.at[pl.ds(brow, nblk)], idxs_v)
            for p in range(nch):
                for q in range(8):
                    pltpu.sync_copy(
                        zb_v, a_sh.at[pl.ds(r0 + q * piece, piece)])
                plsc.subcore_barrier()

                def blk(b, carry):
                    base = e0 + b * _EB
                    cp = pltpu.async_copy(
                        relh_hbm.at[pl.ds(base, _EB), pl.ds(p * wc, wc)],
                        rh_v, sem1)
                    cp.wait()
                    pltpu.sync_copy(rh_v, a_sh.at[idxs_v.at[b]], add=True)
                    return carry
                lax.fori_loop(0, nblk, blk, 0)
                plsc.subcore_barrier()
                for q in range(8):
                    pltpu.sync_copy(
                        a_sh.at[pl.ds(r0 + q * piece, piece)],
                        out_hbm.at[pl.ds(r0 + q * piece, piece),
                                   pl.ds(p * wc, wc)])
                plsc.subcore_barrier()

        @pl.when(c == 0)
        def _():
            run(sub3_hbm, ss_hbm)

        @pl.when(c == 1)
        def _():
            run(obj3_hbm, so_hbm)

    return pl.kernel(
        body,
        out_type=[
            jax.ShapeDtypeStruct((np_, h), jnp.float32),
            jax.ShapeDtypeStruct((np_, h), jnp.float32),
        ],
        mesh=_sc_mesh(),
        compiler_params=_SC_PARAMS,
        scratch_types=[
            pltpu.VMEM((epc // _EB, _EB), jnp.int32),
            pltpu.VMEM((_EB, wc), jnp.float32),
            pltpu.VMEM((piece, wc), jnp.float32),
            pltpu.VMEM_SHARED((np_, wc), jnp.float32),
            pltpu.SemaphoreType.DMA,
        ],
    )


def kernel(roi_features, union_features, rel_pair_idxs, obj_labels,
           W_obj_in, W_rel_in, W_s2r, W_o2r, W_r2s, W_r2o,
           W_obj_cls, b_obj, W_rel_cls, b_rel, freq_table):
    n, h = roi_features.shape[0], W_obj_in.shape[1]
    e = union_features.shape[0]
    nobj = W_obj_cls.shape[1]
    nrel = W_rel_cls.shape[1]
    iters = 3

    wc = 128
    nch = h // wc
    np_ = ((n + 1 + _NW * _L - 1) // (_NW * _L)) * (_NW * _L)
    epw = _cdiv(e, _NW * _EB) * _EB
    ep = epw * _NW
    nblk = epw // _EB
    nbias = 128

    sub_p = jnp.concatenate(
        [rel_pair_idxs[:, 0], jnp.full((ep - e,), n, jnp.int32)])
    obj_p = jnp.concatenate(
        [rel_pair_idxs[:, 1], jnp.full((ep - e,), n, jnp.int32)])
    labels_p = jnp.concatenate(
        [obj_labels, jnp.zeros((np_ - n,), jnp.int32)])
    freq_p = jnp.pad(freq_table, ((0, 7), (0, nbias - nrel)))
    w_obj_cls_p = jnp.pad(W_obj_cls, ((0, 0), (0, 256 - nobj)))
    b_obj_p = jnp.pad(b_obj, (0, 256 - nobj)).reshape(1, 256)
    w_rel_cls_p = jnp.pad(W_rel_cls, ((0, 0), (0, nbias - nrel)))
    b_rel_p = jnp.pad(b_rel, (0, nbias - nrel)).reshape(1, nbias)

    d_s, d_o, fbias = _dega_build(np_, ep, epw, nblk, nobj, nbias)(
        sub_p, obj_p, labels_p, freq_p)
    inv_ds, inv_do = _inv_build(np_)(d_s, d_o)

    obj_h = _mm(roi_features, W_obj_in, relu=True)
    rel_h = _mm(union_features, W_rel_in, out_rows=ep, relu=True)

    w_s2r_c = W_s2r.reshape(h, nch, wc).transpose(1, 0, 2)
    w_o2r_c = W_o2r.reshape(h, nch, wc).transpose(1, 0, 2)
    b1 = _b1_build(np_, ep, epw, nblk, h, nch, wc)
    b2 = _b2_build(np_, ep, h, nch, wc)
    ids = inv_ds.reshape(np_, 1)
    ido = inv_do.reshape(np_, 1)
    sub3 = sub_p.reshape(_NW * nblk, _EB)
    obj3 = obj_p.reshape(_NW * nblk, _EB)
    for _ in range(iters):
        p_s, p_o = _proj(obj_h, w_s2r_c, w_o2r_c, np_, nch, wc)
        rel_h = b1(p_s, p_o, rel_h, sub3, obj3)
        s_s, s_o = b2(rel_h, sub3, obj3)
        obj_h = _objupd(obj_h, s_s, s_o, ids, ido, W_r2s, W_r2o)

    obj_logits = _mm(obj_h, w_obj_cls_p, bias=b_obj_p)[:, :nobj]
    rel_logits = _rellog(rel_h, w_rel_cls_p, b_rel_p, fbias, e)[:, :nrel]
    return obj_logits, rel_logits
